# Initial kernel scaffold; baseline (speedup 1.0000x reference)
import jax
import jax.numpy as jnp
from jax.experimental import pallas as pl


def kernel(params, idx):
    raise NotImplementedError("write your pallas kernel here")


# speedup vs baseline: 1.5490x; 1.5490x over previous
import functools

import jax
import jax.numpy as jnp
from jax import lax
from jax.experimental import pallas as pl
from jax.experimental.pallas import tpu as pltpu
from jax.experimental.pallas import tpu_sc as plsc

T = 2048
C = 768
NH = 12
HD = 64
NE = 8
CAP = 640
FF = 3072
NLAYER = 8
BM = 256
NMB = T // BM
TPAD = T + BM
QB = 512
VB = 512
NEG = -1e30
f32 = jnp.float32


def _sc_gather(table, idx):
    B = idx.shape[0]
    D = table.shape[-1]
    info = plsc.get_sparse_core_info()
    nw = info.num_cores * info.num_subcores
    bpw = B // nw
    nch = -(-bpw // 128)
    ck = bpw // nch
    mesh = plsc.VectorSubcoreMesh(core_axis_name="c", subcore_axis_name="s")
    scratch = ([pltpu.VMEM((ck,), jnp.int32) for _ in range(nch)]
               + [pltpu.VMEM((ck, D), table.dtype) for _ in range(nch)]
               + [pltpu.SemaphoreType.DMA])

    @functools.partial(
        pl.kernel, mesh=mesh,
        out_type=jax.ShapeDtypeStruct((B, D), table.dtype),
        scratch_types=scratch)
    def k(table_h, idx_h, out_h, *sc):
        idx_vs = sc[:nch]
        row_vs = sc[nch:2 * nch]
        sem = sc[2 * nch]
        wid = lax.axis_index("s") * info.num_cores + lax.axis_index("c")
        base = wid * bpw
        for j in range(nch):
            pltpu.sync_copy(idx_h.at[pl.ds(base + j * ck, ck)], idx_vs[j])
        copies = [pltpu.async_copy(table_h.at[idx_vs[j]], row_vs[j], sem)
                  for j in range(nch)]
        for cp in copies:
            cp.wait()
        for j in range(nch):
            pltpu.sync_copy(row_vs[j], out_h.at[pl.ds(base + j * ck, ck)])

    return k(table, idx)


def _ln(x, w, b):
    mu = jnp.mean(x, axis=-1, keepdims=True)
    xc = x - mu
    var = jnp.mean(xc * xc, axis=-1, keepdims=True)
    return xc * lax.rsqrt(var + 1e-5) * w + b


_VEC = pl.BlockSpec((1, C), lambda *a: (0, 0))


def _add_ln(a, b2, lnw, lnb):
    def body(a_ref, b_ref, w_ref, bb_ref, x_ref, ln_ref):
        x = a_ref[...] + b_ref[...]
        x_ref[...] = x
        ln_ref[...] = _ln(x, w_ref[...], bb_ref[...])

    bs = pl.BlockSpec((BM, C), lambda i: (i, 0))
    return pl.pallas_call(
        body, grid=(NMB,),
        in_specs=[bs, bs, _VEC, _VEC],
        out_specs=[bs, bs],
        out_shape=[jax.ShapeDtypeStruct((T, C), f32)] * 2,
    )(a, b2, lnw, lnb)


def _mm_bias(x, w, b, bn):
    M, K = x.shape
    N = w.shape[0]

    def body(x_ref, w_ref, b_ref, o_ref):
        o_ref[...] = lax.dot_general(
            x_ref[...], w_ref[...], (((1,), (1,)), ((), ())),
            preferred_element_type=f32) + b_ref[...]

    return pl.pallas_call(
        body, grid=(N // bn, M // BM),
        in_specs=[pl.BlockSpec((BM, K), lambda n, m: (m, 0)),
                  pl.BlockSpec((bn, K), lambda n, m: (n, 0)),
                  pl.BlockSpec((1, bn), lambda n, m: (0, n))],
        out_specs=pl.BlockSpec((BM, bn), lambda n, m: (m, n)),
        out_shape=jax.ShapeDtypeStruct((M, N), f32),
    )(x, w, b)


def _attention(qkv):
    def body(q_ref, k_ref, v_ref, o_ref):
        qb = pl.program_id(0)
        row = qb * QB + lax.broadcasted_iota(jnp.int32, (QB, T), 0)
        col = lax.broadcasted_iota(jnp.int32, (QB, T), 1)
        causal = col <= row
        for h in range(NH):
            q = q_ref[:, h * HD:(h + 1) * HD]
            k = k_ref[:, h * HD:(h + 1) * HD]
            v = v_ref[:, h * HD:(h + 1) * HD]
            s = lax.dot_general(q, k, (((1,), (1,)), ((), ())),
                                preferred_element_type=f32) * 0.125
            s = jnp.where(causal, s, NEG)
            m = jnp.max(s, axis=1, keepdims=True)
            p = jnp.exp(s - m)
            l = jnp.sum(p, axis=1, keepdims=True)
            o_ref[:, h * HD:(h + 1) * HD] = (
                jnp.dot(p, v, preferred_element_type=f32) / l)

    return pl.pallas_call(
        body, grid=(T // QB,),
        in_specs=[pl.BlockSpec((QB, C), lambda qb: (qb, 0)),
                  pl.BlockSpec((T, C), lambda qb: (0, 1)),
                  pl.BlockSpec((T, C), lambda qb: (0, 2))],
        out_specs=pl.BlockSpec((QB, C), lambda qb: (qb, 0)),
        out_shape=jax.ShapeDtypeStruct((T, C), f32),
    )(qkv, qkv, qkv)


def _mm_res_ln(h, w, b, resid, lnw, lnb, pad):
    K = h.shape[1]
    gm = NMB + 1 if pad else NMB
    Mo = TPAD if pad else T
    im = lambda i: (jnp.minimum(i, NMB - 1), 0)

    def body(h_ref, w_ref, b_ref, r_ref, lw_ref, lb_ref, x_ref, ln_ref):
        @pl.when(pl.program_id(0) == NMB)
        def _():
            x_ref[...] = jnp.zeros_like(x_ref)
            ln_ref[...] = jnp.zeros_like(ln_ref)

        @pl.when(pl.program_id(0) < NMB)
        def _():
            x = r_ref[...] + lax.dot_general(
                h_ref[...], w_ref[...], (((1,), (1,)), ((), ())),
                preferred_element_type=f32) + b_ref[...]
            x_ref[...] = x
            ln_ref[...] = _ln(x, lw_ref[...], lb_ref[...])

    bs_o = pl.BlockSpec((BM, C), lambda i: (i, 0))
    return pl.pallas_call(
        body, grid=(gm,),
        in_specs=[pl.BlockSpec((BM, K), im),
                  pl.BlockSpec((C, K), lambda i: (0, 0)),
                  _VEC,
                  pl.BlockSpec((BM, C), im),
                  _VEC, _VEC],
        out_specs=[bs_o, bs_o],
        out_shape=[jax.ShapeDtypeStruct((Mo, C), f32)] * 2,
    )(h, w, b, resid, lnw, lnb)


def _dense_mlp(xln, fcw, fcb, pjw, pjb, resid, lnw, lnb):
    def body(x_ref, fw_ref, fb_ref, pw_ref, pb_ref, r_ref, lw_ref, lb_ref,
             xo_ref, ln_ref):
        h = lax.dot_general(x_ref[...], fw_ref[...], (((1,), (1,)), ((), ())),
                            preferred_element_type=f32) + fb_ref[...]
        h = jax.nn.gelu(h, approximate=True)
        y = lax.dot_general(h, pw_ref[...], (((1,), (1,)), ((), ())),
                            preferred_element_type=f32) + pb_ref[...]
        x = r_ref[...] + y
        xo_ref[...] = x
        ln_ref[...] = _ln(x, lw_ref[...], lb_ref[...])

    bs = pl.BlockSpec((BM, C), lambda i: (i, 0))
    return pl.pallas_call(
        body, grid=(NMB,),
        in_specs=[bs,
                  pl.BlockSpec((FF, C), lambda i: (0, 0)),
                  pl.BlockSpec((1, FF), lambda i: (0, 0)),
                  pl.BlockSpec((C, FF), lambda i: (0, 0)),
                  _VEC, bs, _VEC, _VEC],
        out_specs=[bs, bs],
        out_shape=[jax.ShapeDtypeStruct((T, C), f32)] * 2,
    )(xln, fcw, fcb, pjw, pjb, resid, lnw, lnb)


def _router(xln_pad, wg):
    def body(x_ref, wg_ref, w_ref, s_ref, tok_ref):
        logits = lax.dot_general(x_ref[...], wg_ref[...],
                                 (((1,), (1,)), ((), ())),
                                 preferred_element_type=f32)
        ei = lax.broadcasted_iota(jnp.int32, (T, NE), 1)
        m1 = jnp.max(logits, axis=1, keepdims=True)
        i1 = jnp.min(jnp.where(logits == m1, ei, NE), axis=1, keepdims=True)
        sel1 = ei == i1
        l2 = jnp.where(sel1, NEG, logits)
        m2 = jnp.max(l2, axis=1, keepdims=True)
        i2 = jnp.min(jnp.where(l2 == m2, ei, NE), axis=1, keepdims=True)
        sel2 = ei == i2
        e2 = jnp.exp(m2 - m1)
        denom = 1.0 + e2
        w1 = 1.0 / denom
        w2 = e2 / denom
        s1f = sel1.astype(f32)
        s2f = sel2.astype(f32)
        M = jnp.concatenate([s1f, s2f], axis=1)
        ri = lax.broadcasted_iota(jnp.int32, (T, T), 0)
        ci = lax.broadcasted_iota(jnp.int32, (T, T), 1)
        Lt = (ci <= ri).astype(f32)
        Cm = jnp.dot(Lt, M, preferred_element_type=f32)
        C0 = Cm[:, :NE]
        C1 = Cm[:, NE:]
        cnt0 = jnp.sum(s1f, axis=0, keepdims=True)
        r0 = jnp.sum(s1f * C0, axis=1, keepdims=True) - 1.0
        r1 = jnp.sum(s2f * (C1 + cnt0), axis=1, keepdims=True) - 1.0
        keep0 = r0 < CAP
        keep1 = r1 < CAP
        w1f = jnp.where(keep0, w1, 0.0)
        w2f = jnp.where(keep1, w2, 0.0)
        slot0 = jnp.where(keep0, i1.astype(f32) * CAP + r0, 0.0)
        slot1 = jnp.where(keep1, i2.astype(f32) * CAP + r1, 0.0)
        cap_i = lax.broadcasted_iota(jnp.int32, (T, CAP), 1).astype(f32)
        oh0 = (r0 == cap_i).astype(f32)
        oh1 = (r1 == cap_i).astype(f32)
        tv = lax.broadcasted_iota(jnp.int32, (T, 1), 0).astype(f32)
        dn = (((0,), (0,)), ((), ()))
        hi = lax.Precision.HIGHEST
        T0 = lax.dot_general(s1f * tv, oh0, dn, precision=hi,
                             preferred_element_type=f32)
        T1 = lax.dot_general(s2f * tv, oh1, dn, precision=hi,
                             preferred_element_type=f32)
        F0 = lax.dot_general(s1f, oh0, dn, precision=hi,
                             preferred_element_type=f32)
        F1 = lax.dot_general(s2f, oh1, dn, precision=hi,
                             preferred_element_type=f32)
        tok = T0 + T1 + (1.0 - F0 - F1) * T
        tok_ref[...] = tok.astype(jnp.int32)
        w_ref[...] = jnp.concatenate([w1f, w2f], axis=1)
        s_ref[...] = jnp.concatenate([slot0, slot1], axis=1).astype(jnp.int32)

    return pl.pallas_call(
        body, grid=(1,),
        in_specs=[pl.BlockSpec((T, C), lambda i: (0, 0)),
                  pl.BlockSpec((NE, C), lambda i: (0, 0))],
        out_specs=[pl.BlockSpec((T, 2), lambda i: (0, 0)),
                   pl.BlockSpec((T, 2), lambda i: (0, 0)),
                   pl.BlockSpec((NE, CAP), lambda i: (0, 0))],
        out_shape=[jax.ShapeDtypeStruct((T, 2), f32),
                   jax.ShapeDtypeStruct((T, 2), jnp.int32),
                   jax.ShapeDtypeStruct((NE, CAP), jnp.int32)],
    )(xln_pad, wg)


def _expert_ffn(disp, fcw, fcb, pjw, pjb):
    def body(d_ref, fw_ref, fb_ref, pw_ref, pb_ref, o_ref):
        h = jnp.dot(d_ref[...], fw_ref[0], preferred_element_type=f32)
        h = h + fb_ref[0]
        h = 0.5 * h * (1.0 + lax.erf(h * 0.7071067811865476))
        o_ref[...] = jnp.dot(h, pw_ref[0], preferred_element_type=f32) + pb_ref[0]

    return pl.pallas_call(
        body, grid=(NE,),
        in_specs=[pl.BlockSpec((CAP, C), lambda e: (e, 0)),
                  pl.BlockSpec((1, C, FF), lambda e: (e, 0, 0)),
                  pl.BlockSpec((1, 1, FF), lambda e: (e, 0, 0)),
                  pl.BlockSpec((1, FF, C), lambda e: (e, 0, 0)),
                  pl.BlockSpec((1, 1, C), lambda e: (e, 0, 0))],
        out_specs=pl.BlockSpec((CAP, C), lambda e: (e, 0)),
        out_shape=jax.ShapeDtypeStruct((NE * CAP, C), f32),
    )(disp, fcw, fcb, pjw, pjb)


def _combine_ln(x2pad, g, w2, lnw, lnb):
    def body(x_ref, g0_ref, g1_ref, w_ref, lw_ref, lb_ref, xo_ref, ln_ref):
        w = w_ref[...]
        x = x_ref[...] + w[:, 0:1] * g0_ref[...] + w[:, 1:2] * g1_ref[...]
        xo_ref[...] = x
        ln_ref[...] = _ln(x, lw_ref[...], lb_ref[...])

    bs = pl.BlockSpec((BM, C), lambda i: (i, 0))
    return pl.pallas_call(
        body, grid=(NMB,),
        in_specs=[bs, bs,
                  pl.BlockSpec((BM, C), lambda i: (NMB + i, 0)),
                  pl.BlockSpec((BM, 2), lambda i: (i, 0)),
                  _VEC, _VEC],
        out_specs=[bs, bs],
        out_shape=[jax.ShapeDtypeStruct((T, C), f32)] * 2,
    )(x2pad, g, g, w2, lnw, lnb)


def _lm_head(lnf, wte):
    V = wte.shape[0]

    def body(x_ref, w_ref, o_ref):
        o_ref[...] = lax.dot_general(x_ref[...], w_ref[...],
                                     (((1,), (1,)), ((), ())),
                                     preferred_element_type=f32)

    return pl.pallas_call(
        body, grid=(pl.cdiv(V, VB),),
        in_specs=[pl.BlockSpec((8, C), lambda n: (T // 8 - 1, 0)),
                  pl.BlockSpec((VB, C), lambda n: (n, 0))],
        out_specs=pl.BlockSpec((8, VB), lambda n: (0, n)),
        out_shape=jax.ShapeDtypeStruct((8, V), f32),
    )(lnf, wte)


def kernel(params, idx):
    p = params
    blocks = p['blocks']
    idxf = idx.reshape(T)
    emb = _sc_gather(p['wte'], idxf)
    b0 = blocks[0]
    x, ln1 = _add_ln(emb, p['wpe'][:T],
                     b0['ln1_w'].reshape(1, C), b0['ln1_b'].reshape(1, C))
    for i, blk in enumerate(blocks):
        qkv = _mm_bias(ln1, blk['attn_qkv_w'],
                       blk['attn_qkv_b'].reshape(1, 3 * C), C)
        att = _attention(qkv)
        moe = (i % 2) == 0
        x2, ln2 = _mm_res_ln(att, blk['attn_proj_w'],
                             blk['attn_proj_b'].reshape(1, C), x,
                             blk['ln2_w'].reshape(1, C),
                             blk['ln2_b'].reshape(1, C), pad=moe)
        if moe:
            nxt = blocks[i + 1]
            ws, ss, tok = _router(ln2, blk['router_wg'])
            disp = _sc_gather(ln2, tok.reshape(NE * CAP))
            h = _expert_ffn(disp, blk['exp_fc'], blk['exp_fc_b'],
                            blk['exp_proj'], blk['exp_proj_b'])
            sidx = jnp.concatenate([ss[:, 0], ss[:, 1]], axis=0)
            g = _sc_gather(h, sidx)
            x, ln1 = _combine_ln(x2, g, ws,
                                 nxt['ln1_w'].reshape(1, C),
                                 nxt['ln1_b'].reshape(1, C))
        else:
            if i + 1 < NLAYER:
                nlw, nlb = blocks[i + 1]['ln1_w'], blocks[i + 1]['ln1_b']
            else:
                nlw, nlb = p['ln_f_w'], p['ln_f_b']
            x, ln1 = _dense_mlp(ln2, blk['mlp_fc_w'],
                                blk['mlp_fc_b'].reshape(1, FF),
                                blk['mlp_proj_w'],
                                blk['mlp_proj_b'].reshape(1, C),
                                x2, nlw.reshape(1, C), nlb.reshape(1, C))
    logits8 = _lm_head(ln1, p['wte'])
    return logits8[7:8, :][None, :, :]
